# baseline (speedup 1.0000x reference)
import functools

import jax
import jax.numpy as jnp
from jax import lax
from jax.experimental import pallas as pl
from jax.experimental.pallas import tpu as pltpu
from jax.experimental.pallas import tpu_sc as plsc

N_NODES = 10000
N_EDGES = 320000
F = 128

NC = 2
NS = 16
NW = NC * NS
B = 128

E_PAD = ((N_EDGES + 2 * NW * B - 1) // (2 * NW * B)) * (2 * NW * B)
NB = E_PAD // (NW * B)
NBH = NB // 2
N_ACC = ((N_NODES + 1 + NS * 128 - 1) // (NS * 128)) * (NS * 128)
TROWS = N_ACC // NS
NBC = 40

_mesh = plsc.VectorSubcoreMesh(
    core_axis_name="c", subcore_axis_name="s", num_cores=NC, num_subcores=NS
)


@functools.partial(
    pl.kernel,
    out_type=(
        jax.ShapeDtypeStruct((NC * N_ACC,), jnp.float32),
        jax.ShapeDtypeStruct((NW, NB, B), jnp.int32),
    ),
    mesh=_mesh,
    scratch_types=[
        pltpu.VMEM((NB, B), jnp.int32),
        pltpu.VMEM((NB, B), jnp.int32),
        pltpu.VMEM((B,), jnp.float32),
        pltpu.VMEM((TROWS,), jnp.float32),
        pltpu.VMEM_SHARED((N_ACC,), jnp.float32),
    ],
)
def _sc_deg(row_hbm, col_hbm, deg_hbm, dest_hbm, row_v, dest_v, ones_v, zv, deg_sh):
    c = lax.axis_index("c")
    s = lax.axis_index("s")
    w = c * NS + s

    pltpu.sync_copy(row_hbm.at[w], row_v)
    pltpu.sync_copy(col_hbm.at[w], dest_v)

    one16 = jnp.ones((16,), jnp.float32)
    zero16 = jnp.zeros((16,), jnp.float32)
    for g in range(B // 16):
        ones_v[pl.ds(g * 16, 16)] = one16

    def zfill(k, carry):
        zv[pl.ds(k * 16, 16)] = zero16
        return carry

    lax.fori_loop(0, TROWS // 16, zfill, 0)
    pltpu.sync_copy(zv, deg_sh.at[pl.ds(s * TROWS, TROWS)])

    dummy = jnp.full((16,), N_NODES, jnp.int32)

    def mask_body(j, carry):
        for g in range(B // 16):
            r = row_v[j, pl.ds(g * 16, 16)]
            cc = dest_v[j, pl.ds(g * 16, 16)]
            dest_v[j, pl.ds(g * 16, 16)] = jnp.where(r == cc, dummy, r)
        return carry

    lax.fori_loop(0, NB, mask_body, 0)
    pltpu.sync_copy(dest_v, dest_hbm.at[w])

    plsc.subcore_barrier()

    def add_body(j, carry):
        pltpu.sync_copy(ones_v, deg_sh.at[dest_v.at[j]], add=True)
        return carry

    lax.fori_loop(0, NB, add_body, 0)

    plsc.subcore_barrier()
    pltpu.sync_copy(
        deg_sh.at[pl.ds(s * TROWS, TROWS)],
        deg_hbm.at[pl.ds(c * N_ACC + s * TROWS, TROWS)],
    )


@functools.partial(
    pl.kernel,
    out_type=jax.ShapeDtypeStruct((NC * N_ACC, F), jnp.float32),
    mesh=_mesh,
    scratch_types=[
        pltpu.VMEM((NB, B), jnp.int32),
        pltpu.VMEM((NB, B), jnp.int32),
        pltpu.VMEM((B, F), jnp.float32),
        pltpu.VMEM_SHARED((N_ACC, F), jnp.float32),
        pltpu.SemaphoreType.DMA,
    ],
)
def _sc_spmm(h2_hbm, col_hbm, dest_hbm, zinit_hbm, acc_hbm,
             col_v, dest_v, gb0, acc_sh, gsem):
    c = lax.axis_index("c")
    s = lax.axis_index("s")
    w = c * NS + s

    pltpu.sync_copy(col_hbm.at[w], col_v)
    pltpu.sync_copy(dest_hbm.at[w], dest_v)
    pltpu.sync_copy(zinit_hbm, acc_sh.at[pl.ds(s * TROWS, TROWS)])
    plsc.subcore_barrier()

    def body(j, carry):
        pltpu.async_copy(h2_hbm.at[col_v.at[j]], gb0, gsem).wait()
        pltpu.sync_copy(gb0, acc_sh.at[dest_v.at[j]], add=True)
        return carry

    lax.fori_loop(0, NB, body, 0)

    plsc.subcore_barrier()
    pltpu.sync_copy(
        acc_sh.at[pl.ds(s * TROWS, TROWS)],
        acc_hbm.at[pl.ds(c * N_ACC + s * TROWS, TROWS)],
    )


_RB = 1000


def _tc_h2_body(x_ref, w_ref, deg_ref, h2_ref):
    h = jnp.dot(x_ref[...], w_ref[...], preferred_element_type=jnp.float32)
    deg = deg_ref[:, 0] + deg_ref[:, 1] + 1.0
    dis = lax.rsqrt(deg)
    h2_ref[...] = h * dis[:, None]


_tc_h2 = pl.pallas_call(
    _tc_h2_body,
    grid=(N_NODES // _RB,),
    in_specs=[
        pl.BlockSpec((_RB, F), lambda i: (i, 0)),
        pl.BlockSpec((F, F), lambda i: (0, 0)),
        pl.BlockSpec((_RB, NC), lambda i: (i, 0)),
    ],
    out_specs=pl.BlockSpec((_RB, F), lambda i: (i, 0)),
    out_shape=jax.ShapeDtypeStruct((N_NODES, F), jnp.float32),
)


def _tc_out_body(acc_ref, h2_ref, deg_ref, b_ref, o_ref):
    acc = acc_ref[0] + acc_ref[1]
    deg = deg_ref[:, 0] + deg_ref[:, 1] + 1.0
    dis = lax.rsqrt(deg)
    o_ref[...] = (acc + h2_ref[...]) * dis[:, None] + b_ref[...]


_tc_out = pl.pallas_call(
    _tc_out_body,
    grid=(N_NODES // _RB,),
    in_specs=[
        pl.BlockSpec((NC, _RB, F), lambda i: (0, i, 0)),
        pl.BlockSpec((_RB, F), lambda i: (i, 0)),
        pl.BlockSpec((_RB, NC), lambda i: (i, 0)),
        pl.BlockSpec((1, F), lambda i: (0, 0)),
    ],
    out_specs=pl.BlockSpec((_RB, F), lambda i: (i, 0)),
    out_shape=jax.ShapeDtypeStruct((N_NODES, F), jnp.float32),
)


def kernel(x, edge_index, weight, bias):
    assert x.shape == (N_NODES, F) and edge_index.shape == (2, N_EDGES)
    row = edge_index[0]
    col = edge_index[1]
    pad = E_PAD - N_EDGES
    zpad = jnp.zeros((pad,), jnp.int32)
    row_p = jnp.concatenate([row, zpad]).reshape(NW, NB, B)
    col_p = jnp.concatenate([col, zpad]).reshape(NW, NB, B)

    deg_parts, dest = _sc_deg(row_p, col_p)
    deg_parts = deg_parts.reshape(NC, N_ACC)
    deg2 = jnp.stack([deg_parts[0, :N_NODES], deg_parts[1, :N_NODES]], axis=1)

    h2 = _tc_h2(x, weight, deg2)

    zinit = jnp.zeros((TROWS, F), jnp.float32)
    acc_parts = _sc_spmm(h2, col_p, dest, zinit).reshape(NC, N_ACC, F)

    out = _tc_out(acc_parts, h2, deg2, bias.reshape(1, F))
    return out


# speedup vs baseline: 2.5500x; 2.5500x over previous
import functools

import jax
import jax.numpy as jnp
from jax import lax
from jax.experimental import pallas as pl
from jax.experimental.pallas import tpu as pltpu
from jax.experimental.pallas import tpu_sc as plsc

N_NODES = 10000
N_EDGES = 320000
F = 128

NC = 2
NS = 16
NW = NC * NS
B = 128

E_PAD = ((N_EDGES + 2 * NW * B - 1) // (2 * NW * B)) * (2 * NW * B)
NB = E_PAD // (NW * B)
NBH = NB // 2
N_ACC = ((N_NODES + 1 + NS * 128 - 1) // (NS * 128)) * (NS * 128)
TROWS = N_ACC // NS
NBC = 40

_mesh = plsc.VectorSubcoreMesh(
    core_axis_name="c", subcore_axis_name="s", num_cores=NC, num_subcores=NS
)


@functools.partial(
    pl.kernel,
    out_type=(
        jax.ShapeDtypeStruct((NC * N_ACC,), jnp.float32),
        jax.ShapeDtypeStruct((NW, NB, B), jnp.int32),
    ),
    mesh=_mesh,
    scratch_types=[
        pltpu.VMEM((NB, B), jnp.int32),
        pltpu.VMEM((NB, B), jnp.int32),
        pltpu.VMEM((B,), jnp.float32),
        pltpu.VMEM((TROWS,), jnp.float32),
        pltpu.VMEM_SHARED((N_ACC,), jnp.float32),
    ],
)
def _sc_deg(row_hbm, col_hbm, deg_hbm, dest_hbm, row_v, dest_v, ones_v, zv, deg_sh):
    c = lax.axis_index("c")
    s = lax.axis_index("s")
    w = c * NS + s

    pltpu.sync_copy(row_hbm.at[w], row_v)
    pltpu.sync_copy(col_hbm.at[w], dest_v)

    one16 = jnp.ones((16,), jnp.float32)
    zero16 = jnp.zeros((16,), jnp.float32)
    for g in range(B // 16):
        ones_v[pl.ds(g * 16, 16)] = one16

    def zfill(k, carry):
        zv[pl.ds(k * 16, 16)] = zero16
        return carry

    lax.fori_loop(0, TROWS // 16, zfill, 0)
    pltpu.sync_copy(zv, deg_sh.at[pl.ds(s * TROWS, TROWS)])

    nsplat = jnp.full((16,), N_NODES, jnp.int32)
    m127 = jnp.full((16,), 127, jnp.int32)

    def mask_body(j, carry):
        for g in range(B // 16):
            r = row_v[j, pl.ds(g * 16, 16)]
            cc = dest_v[j, pl.ds(g * 16, 16)]
            dest_v[j, pl.ds(g * 16, 16)] = jnp.where(r == cc, nsplat + (r & m127), r)
        return carry

    lax.fori_loop(0, NB, mask_body, 0)
    pltpu.sync_copy(dest_v, dest_hbm.at[w])

    plsc.subcore_barrier()

    def add_body(j, carry):
        pltpu.sync_copy(ones_v, deg_sh.at[dest_v.at[j]], add=True)
        return carry

    lax.fori_loop(0, NB, add_body, 0)

    plsc.subcore_barrier()
    pltpu.sync_copy(
        deg_sh.at[pl.ds(s * TROWS, TROWS)],
        deg_hbm.at[pl.ds(c * N_ACC + s * TROWS, TROWS)],
    )


@functools.partial(
    pl.kernel,
    out_type=jax.ShapeDtypeStruct((NC * N_ACC, F), jnp.float32),
    mesh=_mesh,
    scratch_types=[
        pltpu.VMEM((NB, B), jnp.int32),
        pltpu.VMEM((NB, B), jnp.int32),
        pltpu.VMEM((B, F), jnp.float32),
        pltpu.VMEM_SHARED((N_ACC, F), jnp.float32),
        pltpu.SemaphoreType.DMA,
    ],
)
def _sc_spmm(h2_hbm, col_hbm, dest_hbm, zinit_hbm, acc_hbm,
             col_v, dest_v, gb0, acc_sh, gsem):
    c = lax.axis_index("c")
    s = lax.axis_index("s")
    w = c * NS + s

    pltpu.sync_copy(col_hbm.at[w], col_v)
    pltpu.sync_copy(dest_hbm.at[w], dest_v)
    pltpu.sync_copy(zinit_hbm, acc_sh.at[pl.ds(s * TROWS, TROWS)])
    plsc.subcore_barrier()

    def body(j, carry):
        pltpu.async_copy(h2_hbm.at[col_v.at[j]], gb0, gsem).wait()
        pltpu.sync_copy(gb0, acc_sh.at[dest_v.at[j]], add=True)
        return carry

    lax.fori_loop(0, NB, body, 0)

    plsc.subcore_barrier()
    pltpu.sync_copy(
        acc_sh.at[pl.ds(s * TROWS, TROWS)],
        acc_hbm.at[pl.ds(c * N_ACC + s * TROWS, TROWS)],
    )


_RB = 1000


def _tc_h2_body(x_ref, w_ref, deg_ref, h2_ref):
    h = jnp.dot(x_ref[...], w_ref[...], preferred_element_type=jnp.float32)
    deg = deg_ref[:, 0] + deg_ref[:, 1] + 1.0
    dis = lax.rsqrt(deg)
    h2_ref[...] = h * dis[:, None]


_tc_h2 = pl.pallas_call(
    _tc_h2_body,
    grid=(N_NODES // _RB,),
    in_specs=[
        pl.BlockSpec((_RB, F), lambda i: (i, 0)),
        pl.BlockSpec((F, F), lambda i: (0, 0)),
        pl.BlockSpec((_RB, NC), lambda i: (i, 0)),
    ],
    out_specs=pl.BlockSpec((_RB, F), lambda i: (i, 0)),
    out_shape=jax.ShapeDtypeStruct((N_NODES, F), jnp.float32),
)


def _tc_out_body(acc_ref, h2_ref, deg_ref, b_ref, o_ref):
    acc = acc_ref[0] + acc_ref[1]
    deg = deg_ref[:, 0] + deg_ref[:, 1] + 1.0
    dis = lax.rsqrt(deg)
    o_ref[...] = (acc + h2_ref[...]) * dis[:, None] + b_ref[...]


_tc_out = pl.pallas_call(
    _tc_out_body,
    grid=(N_NODES // _RB,),
    in_specs=[
        pl.BlockSpec((NC, _RB, F), lambda i: (0, i, 0)),
        pl.BlockSpec((_RB, F), lambda i: (i, 0)),
        pl.BlockSpec((_RB, NC), lambda i: (i, 0)),
        pl.BlockSpec((1, F), lambda i: (0, 0)),
    ],
    out_specs=pl.BlockSpec((_RB, F), lambda i: (i, 0)),
    out_shape=jax.ShapeDtypeStruct((N_NODES, F), jnp.float32),
)


def kernel(x, edge_index, weight, bias):
    assert x.shape == (N_NODES, F) and edge_index.shape == (2, N_EDGES)
    row = edge_index[0]
    col = edge_index[1]
    pad = E_PAD - N_EDGES
    zpad = jnp.arange(pad, dtype=jnp.int32) & 127
    row_p = jnp.concatenate([row, zpad]).reshape(NW, NB, B)
    col_p = jnp.concatenate([col, zpad]).reshape(NW, NB, B)

    deg_parts, dest = _sc_deg(row_p, col_p)
    deg_parts = deg_parts.reshape(NC, N_ACC)
    deg2 = jnp.stack([deg_parts[0, :N_NODES], deg_parts[1, :N_NODES]], axis=1)

    h2 = _tc_h2(x, weight, deg2)

    zinit = jnp.zeros((TROWS, F), jnp.float32)
    acc_parts = _sc_spmm(h2, col_p, dest, zinit).reshape(NC, N_ACC, F)

    out = _tc_out(acc_parts, h2, deg2, bias.reshape(1, F))
    return out


# speedup vs baseline: 3.1081x; 1.2189x over previous
import functools

import jax
import jax.numpy as jnp
from jax import lax
from jax.experimental import pallas as pl
from jax.experimental.pallas import tpu as pltpu
from jax.experimental.pallas import tpu_sc as plsc

N_NODES = 10000
N_EDGES = 320000
F = 128

NC = 2
NS = 16
NW = NC * NS
B = 128

E_PAD = ((N_EDGES + 2 * NW * B - 1) // (2 * NW * B)) * (2 * NW * B)
NB = E_PAD // (NW * B)
NBH = NB // 2
N_ACC = ((N_NODES + 1 + NS * 128 - 1) // (NS * 128)) * (NS * 128)
TROWS = N_ACC // NS
NBC = 40

_mesh = plsc.VectorSubcoreMesh(
    core_axis_name="c", subcore_axis_name="s", num_cores=NC, num_subcores=NS
)


@functools.partial(
    pl.kernel,
    out_type=(
        jax.ShapeDtypeStruct((NC * N_ACC,), jnp.float32),
        jax.ShapeDtypeStruct((NW, NB, B), jnp.int32),
    ),
    mesh=_mesh,
    scratch_types=[
        pltpu.VMEM((NB, B), jnp.int32),
        pltpu.VMEM((NB, B), jnp.int32),
        pltpu.VMEM((B,), jnp.float32),
        pltpu.VMEM((TROWS,), jnp.float32),
        pltpu.VMEM_SHARED((N_ACC,), jnp.float32),
    ],
)
def _sc_deg(row_hbm, col_hbm, deg_hbm, dest_hbm, row_v, dest_v, ones_v, zv, deg_sh):
    c = lax.axis_index("c")
    s = lax.axis_index("s")
    w = c * NS + s

    pltpu.sync_copy(row_hbm.at[w], row_v)
    pltpu.sync_copy(col_hbm.at[w], dest_v)

    one16 = jnp.ones((16,), jnp.float32)
    zero16 = jnp.zeros((16,), jnp.float32)
    for g in range(B // 16):
        ones_v[pl.ds(g * 16, 16)] = one16

    def zfill(k, carry):
        zv[pl.ds(k * 16, 16)] = zero16
        return carry

    lax.fori_loop(0, TROWS // 16, zfill, 0)
    pltpu.sync_copy(zv, deg_sh.at[pl.ds(s * TROWS, TROWS)])

    nsplat = jnp.full((16,), N_NODES, jnp.int32)
    m127 = jnp.full((16,), 127, jnp.int32)

    def mask_body(j, carry):
        for g in range(B // 16):
            r = row_v[j, pl.ds(g * 16, 16)]
            cc = dest_v[j, pl.ds(g * 16, 16)]
            dest_v[j, pl.ds(g * 16, 16)] = jnp.where(r == cc, nsplat + (r & m127), r)
        return carry

    lax.fori_loop(0, NB, mask_body, 0)
    pltpu.sync_copy(dest_v, dest_hbm.at[w])

    plsc.subcore_barrier()

    def add_body(j, carry):
        pltpu.sync_copy(ones_v, deg_sh.at[dest_v.at[j]], add=True)
        return carry

    lax.fori_loop(0, NB, add_body, 0)

    plsc.subcore_barrier()
    pltpu.sync_copy(
        deg_sh.at[pl.ds(s * TROWS, TROWS)],
        deg_hbm.at[pl.ds(c * N_ACC + s * TROWS, TROWS)],
    )


@functools.partial(
    pl.kernel,
    out_type=jax.ShapeDtypeStruct((NC * N_ACC, F), jnp.float32),
    mesh=_mesh,
    scratch_types=[
        pltpu.VMEM((NBC, B), jnp.int32),
        pltpu.VMEM((NBC, B), jnp.int32),
        pltpu.VMEM((B, F), jnp.float32),
        pltpu.VMEM((B, F), jnp.float32),
        pltpu.VMEM_SHARED((N_ACC, F), jnp.float32),
        pltpu.SemaphoreType.DMA,
    ],
)
def _sc_spmm(h2_hbm, col_hbm, dest_hbm, zinit_hbm, acc_hbm,
             col_v, dest_v, gb0, gb1, acc_sh, gsem):
    c = lax.axis_index("c")
    s = lax.axis_index("s")
    w = c * NS + s

    pltpu.sync_copy(zinit_hbm, acc_sh.at[pl.ds(s * TROWS, TROWS)])
    plsc.subcore_barrier()

    def gather(j, buf):
        return pltpu.make_async_copy(h2_hbm.at[col_v.at[j]], buf, gsem)

    def sscatter(j, buf):
        pltpu.sync_copy(buf, acc_sh.at[dest_v.at[j]], add=True)

    for h in range(NB // NBC):
        pltpu.sync_copy(col_hbm.at[w, pl.ds(h * NBC, NBC)], col_v)
        pltpu.sync_copy(dest_hbm.at[w, pl.ds(h * NBC, NBC)], dest_v)
        gather(0, gb0).start()

        def body(k, carry):
            j0 = 2 * k
            j1 = j0 + 1
            gather(j0, gb0).wait()
            gather(j1, gb1).start()
            sscatter(j0, gb0)
            gather(j1, gb1).wait()
            gather(j0 + 2, gb0).start()
            sscatter(j1, gb1)
            return carry

        lax.fori_loop(0, NBC // 2 - 1, body, 0)
        gather(NBC - 2, gb0).wait()
        gather(NBC - 1, gb1).start()
        sscatter(NBC - 2, gb0)
        gather(NBC - 1, gb1).wait()
        sscatter(NBC - 1, gb1)

    plsc.subcore_barrier()
    pltpu.sync_copy(
        acc_sh.at[pl.ds(s * TROWS, TROWS)],
        acc_hbm.at[pl.ds(c * N_ACC + s * TROWS, TROWS)],
    )


_RB = 1000


def _tc_h2_body(x_ref, w_ref, deg_ref, h2_ref):
    h = jnp.dot(x_ref[...], w_ref[...], preferred_element_type=jnp.float32)
    deg = deg_ref[:, 0] + deg_ref[:, 1] + 1.0
    dis = lax.rsqrt(deg)
    h2_ref[...] = h * dis[:, None]


_tc_h2 = pl.pallas_call(
    _tc_h2_body,
    grid=(N_NODES // _RB,),
    in_specs=[
        pl.BlockSpec((_RB, F), lambda i: (i, 0)),
        pl.BlockSpec((F, F), lambda i: (0, 0)),
        pl.BlockSpec((_RB, NC), lambda i: (i, 0)),
    ],
    out_specs=pl.BlockSpec((_RB, F), lambda i: (i, 0)),
    out_shape=jax.ShapeDtypeStruct((N_NODES, F), jnp.float32),
)


def _tc_out_body(acc_ref, h2_ref, deg_ref, b_ref, o_ref):
    acc = acc_ref[0] + acc_ref[1]
    deg = deg_ref[:, 0] + deg_ref[:, 1] + 1.0
    dis = lax.rsqrt(deg)
    o_ref[...] = (acc + h2_ref[...]) * dis[:, None] + b_ref[...]


_tc_out = pl.pallas_call(
    _tc_out_body,
    grid=(N_NODES // _RB,),
    in_specs=[
        pl.BlockSpec((NC, _RB, F), lambda i: (0, i, 0)),
        pl.BlockSpec((_RB, F), lambda i: (i, 0)),
        pl.BlockSpec((_RB, NC), lambda i: (i, 0)),
        pl.BlockSpec((1, F), lambda i: (0, 0)),
    ],
    out_specs=pl.BlockSpec((_RB, F), lambda i: (i, 0)),
    out_shape=jax.ShapeDtypeStruct((N_NODES, F), jnp.float32),
)


def kernel(x, edge_index, weight, bias):
    assert x.shape == (N_NODES, F) and edge_index.shape == (2, N_EDGES)
    row = edge_index[0]
    col = edge_index[1]
    pad = E_PAD - N_EDGES
    zpad = jnp.arange(pad, dtype=jnp.int32) & 127
    row_p = jnp.concatenate([row, zpad]).reshape(NW, NB, B)
    col_p = jnp.concatenate([col, zpad]).reshape(NW, NB, B)

    deg_parts, dest = _sc_deg(row_p, col_p)
    deg_parts = deg_parts.reshape(NC, N_ACC)
    deg2 = jnp.stack([deg_parts[0, :N_NODES], deg_parts[1, :N_NODES]], axis=1)

    h2 = _tc_h2(x, weight, deg2)

    zinit = jnp.zeros((TROWS, F), jnp.float32)
    acc_parts = _sc_spmm(h2, col_p, dest, zinit).reshape(NC, N_ACC, F)

    out = _tc_out(acc_parts, h2, deg2, bias.reshape(1, F))
    return out


# speedup vs baseline: 3.2716x; 1.0526x over previous
import functools

import jax
import jax.numpy as jnp
import numpy as np
from jax import lax
from jax.experimental import pallas as pl
from jax.experimental.pallas import tpu as pltpu
from jax.experimental.pallas import tpu_sc as plsc

N_NODES = 10000
N_EDGES = 320000
F = 128

NC = 2
NS = 16
NW = NC * NS
B = 128

E_PAD = ((N_EDGES + 2 * NW * B - 1) // (2 * NW * B)) * (2 * NW * B)
NB = E_PAD // (NW * B)
NBH = NB // 2
N_ACC = ((N_NODES + 1 + NS * 128 - 1) // (NS * 128)) * (NS * 128)
TROWS = N_ACC // NS
NBC = 40

_PAD_IDS = (np.arange(E_PAD - N_EDGES, dtype=np.int32) & 127)
_EI_PAD = np.stack([_PAD_IDS, _PAD_IDS])

_mesh = plsc.VectorSubcoreMesh(
    core_axis_name="c", subcore_axis_name="s", num_cores=NC, num_subcores=NS
)


@functools.partial(
    pl.kernel,
    out_type=(
        jax.ShapeDtypeStruct((NC * N_ACC,), jnp.float32),
        jax.ShapeDtypeStruct((NW, NB, B), jnp.int32),
    ),
    mesh=_mesh,
    scratch_types=[
        pltpu.VMEM((NB, B), jnp.int32),
        pltpu.VMEM((NB, B), jnp.int32),
        pltpu.VMEM((B,), jnp.float32),
        pltpu.VMEM((TROWS,), jnp.float32),
        pltpu.VMEM_SHARED((N_ACC,), jnp.float32),
    ],
)
def _sc_deg(ei_hbm, deg_hbm, dest_hbm, row_v, dest_v, ones_v, zv, deg_sh):
    c = lax.axis_index("c")
    s = lax.axis_index("s")
    w = c * NS + s

    pltpu.sync_copy(ei_hbm.at[0, w], row_v)
    pltpu.sync_copy(ei_hbm.at[1, w], dest_v)

    one16 = jnp.ones((16,), jnp.float32)
    zero16 = jnp.zeros((16,), jnp.float32)
    for g in range(B // 16):
        ones_v[pl.ds(g * 16, 16)] = one16

    def zfill(k, carry):
        zv[pl.ds(k * 16, 16)] = zero16
        return carry

    lax.fori_loop(0, TROWS // 16, zfill, 0)
    pltpu.sync_copy(zv, deg_sh.at[pl.ds(s * TROWS, TROWS)])

    nsplat = jnp.full((16,), N_NODES, jnp.int32)
    m127 = jnp.full((16,), 127, jnp.int32)

    def mask_body(j, carry):
        for g in range(B // 16):
            r = row_v[j, pl.ds(g * 16, 16)]
            cc = dest_v[j, pl.ds(g * 16, 16)]
            dest_v[j, pl.ds(g * 16, 16)] = jnp.where(r == cc, nsplat + (r & m127), r)
        return carry

    lax.fori_loop(0, NB, mask_body, 0)
    pltpu.sync_copy(dest_v, dest_hbm.at[w])

    plsc.subcore_barrier()

    def add_body(j, carry):
        pltpu.sync_copy(ones_v, deg_sh.at[dest_v.at[j]], add=True)
        return carry

    lax.fori_loop(0, NB, add_body, 0)

    plsc.subcore_barrier()
    pltpu.sync_copy(
        deg_sh.at[pl.ds(s * TROWS, TROWS)],
        deg_hbm.at[pl.ds(c * N_ACC + s * TROWS, TROWS)],
    )


@functools.partial(
    pl.kernel,
    out_type=jax.ShapeDtypeStruct((NC * N_ACC, F), jnp.float32),
    mesh=_mesh,
    scratch_types=[
        pltpu.VMEM((NBC, B), jnp.int32),
        pltpu.VMEM((NBC, B), jnp.int32),
        pltpu.VMEM((B, F), jnp.float32),
        pltpu.VMEM((B, F), jnp.float32),
        pltpu.VMEM_SHARED((N_ACC, F), jnp.float32),
        pltpu.SemaphoreType.DMA,
    ],
)
def _sc_spmm(h2_hbm, ei_hbm, dest_hbm, zinit_hbm, acc_hbm,
             col_v, dest_v, gb0, gb1, acc_sh, gsem):
    c = lax.axis_index("c")
    s = lax.axis_index("s")
    w = c * NS + s

    pltpu.sync_copy(zinit_hbm, acc_sh.at[pl.ds(s * TROWS, TROWS)])
    plsc.subcore_barrier()

    def gather(j, buf):
        return pltpu.make_async_copy(h2_hbm.at[col_v.at[j]], buf, gsem)

    def sscatter(j, buf):
        pltpu.sync_copy(buf, acc_sh.at[dest_v.at[j]], add=True)

    for h in range(NB // NBC):
        pltpu.sync_copy(ei_hbm.at[1, w, pl.ds(h * NBC, NBC)], col_v)
        pltpu.sync_copy(dest_hbm.at[w, pl.ds(h * NBC, NBC)], dest_v)
        gather(0, gb0).start()

        def body(k, carry):
            j0 = 2 * k
            j1 = j0 + 1
            gather(j0, gb0).wait()
            gather(j1, gb1).start()
            sscatter(j0, gb0)
            gather(j1, gb1).wait()
            gather(j0 + 2, gb0).start()
            sscatter(j1, gb1)
            return carry

        lax.fori_loop(0, NBC // 2 - 1, body, 0)
        gather(NBC - 2, gb0).wait()
        gather(NBC - 1, gb1).start()
        sscatter(NBC - 2, gb0)
        gather(NBC - 1, gb1).wait()
        sscatter(NBC - 1, gb1)

    plsc.subcore_barrier()
    pltpu.sync_copy(
        acc_sh.at[pl.ds(s * TROWS, TROWS)],
        acc_hbm.at[pl.ds(c * N_ACC + s * TROWS, TROWS)],
    )


_RB = 1000


def _tc_h2_body(x_ref, w_ref, deg_ref, h2_ref):
    h = jnp.dot(x_ref[...], w_ref[...], preferred_element_type=jnp.float32)
    deg = deg_ref[:, 0] + deg_ref[:, 1] + 1.0
    dis = lax.rsqrt(deg)
    h2_ref[...] = h * dis[:, None]


_tc_h2 = pl.pallas_call(
    _tc_h2_body,
    grid=(N_NODES // _RB,),
    in_specs=[
        pl.BlockSpec((_RB, F), lambda i: (i, 0)),
        pl.BlockSpec((F, F), lambda i: (0, 0)),
        pl.BlockSpec((_RB, NC), lambda i: (i, 0)),
    ],
    out_specs=pl.BlockSpec((_RB, F), lambda i: (i, 0)),
    out_shape=jax.ShapeDtypeStruct((N_NODES, F), jnp.float32),
)


def _tc_out_body(acc_ref, h2_ref, deg_ref, b_ref, o_ref):
    acc = acc_ref[0] + acc_ref[1]
    deg = deg_ref[:, 0] + deg_ref[:, 1] + 1.0
    dis = lax.rsqrt(deg)
    o_ref[...] = (acc + h2_ref[...]) * dis[:, None] + b_ref[...]


_tc_out = pl.pallas_call(
    _tc_out_body,
    grid=(N_NODES // _RB,),
    in_specs=[
        pl.BlockSpec((NC, _RB, F), lambda i: (0, i, 0)),
        pl.BlockSpec((_RB, F), lambda i: (i, 0)),
        pl.BlockSpec((_RB, NC), lambda i: (i, 0)),
        pl.BlockSpec((1, F), lambda i: (0, 0)),
    ],
    out_specs=pl.BlockSpec((_RB, F), lambda i: (i, 0)),
    out_shape=jax.ShapeDtypeStruct((N_NODES, F), jnp.float32),
)


def kernel(x, edge_index, weight, bias):
    assert x.shape == (N_NODES, F) and edge_index.shape == (2, N_EDGES)
    ei_p = jnp.concatenate([edge_index, _EI_PAD], axis=1).reshape(2, NW, NB, B)

    deg_parts, dest = _sc_deg(ei_p)
    deg_parts = deg_parts.reshape(NC, N_ACC)
    deg2 = jnp.stack([deg_parts[0, :N_NODES], deg_parts[1, :N_NODES]], axis=1)

    h2 = _tc_h2(x, weight, deg2)

    zinit = jnp.zeros((TROWS, F), jnp.float32)
    acc_parts = _sc_spmm(h2, ei_p, dest, zinit).reshape(NC, N_ACC, F)

    out = _tc_out(acc_parts, h2, deg2, bias.reshape(1, F))
    return out


# speedup vs baseline: 3.4706x; 1.0608x over previous
import functools

import jax
import jax.numpy as jnp
import numpy as np
from jax import lax
from jax.experimental import pallas as pl
from jax.experimental.pallas import tpu as pltpu
from jax.experimental.pallas import tpu_sc as plsc

N_NODES = 10000
N_EDGES = 320000
F = 128

NC = 2
NS = 16
NW = NC * NS
B = 112

E_PAD = ((N_EDGES + NW * B - 1) // (NW * B)) * (NW * B)
NB = E_PAD // (NW * B)
N_ACC = ((N_NODES + 1 + NS * 128 - 1) // (NS * 128)) * (NS * 128)
TROWS = N_ACC // NS
NBC = 18
NCH = NB // NBC

_PAD_IDS = (np.arange(E_PAD - N_EDGES, dtype=np.int32) & 127)
_EI_PAD = np.stack([_PAD_IDS, _PAD_IDS])

_mesh = plsc.VectorSubcoreMesh(
    core_axis_name="c", subcore_axis_name="s", num_cores=NC, num_subcores=NS
)


@functools.partial(
    pl.kernel,
    out_type=(
        jax.ShapeDtypeStruct((NC * N_ACC,), jnp.float32),
        jax.ShapeDtypeStruct((NW, NB, B), jnp.int32),
    ),
    mesh=_mesh,
    scratch_types=[
        pltpu.VMEM((NB, B), jnp.int32),
        pltpu.VMEM((NB, B), jnp.int32),
        pltpu.VMEM((B,), jnp.float32),
        pltpu.VMEM((TROWS,), jnp.float32),
        pltpu.VMEM_SHARED((N_ACC,), jnp.float32),
    ],
)
def _sc_deg(ei_hbm, deg_hbm, dest_hbm, row_v, dest_v, ones_v, zv, deg_sh):
    c = lax.axis_index("c")
    s = lax.axis_index("s")
    w = c * NS + s

    pltpu.sync_copy(ei_hbm.at[0, w], row_v)
    pltpu.sync_copy(ei_hbm.at[1, w], dest_v)

    one16 = jnp.ones((16,), jnp.float32)
    zero16 = jnp.zeros((16,), jnp.float32)
    for g in range(B // 16):
        ones_v[pl.ds(g * 16, 16)] = one16

    def zfill(k, carry):
        zv[pl.ds(k * 16, 16)] = zero16
        return carry

    lax.fori_loop(0, TROWS // 16, zfill, 0)
    pltpu.sync_copy(zv, deg_sh.at[pl.ds(s * TROWS, TROWS)])

    nsplat = jnp.full((16,), N_NODES, jnp.int32)
    m127 = jnp.full((16,), 127, jnp.int32)

    def mask_body(j, carry):
        for g in range(B // 16):
            r = row_v[j, pl.ds(g * 16, 16)]
            cc = dest_v[j, pl.ds(g * 16, 16)]
            dest_v[j, pl.ds(g * 16, 16)] = jnp.where(r == cc, nsplat + (r & m127), r)
        return carry

    lax.fori_loop(0, NB, mask_body, 0)
    pltpu.sync_copy(dest_v, dest_hbm.at[w])

    plsc.subcore_barrier()

    def add_body(j, carry):
        pltpu.sync_copy(ones_v, deg_sh.at[dest_v.at[j]], add=True)
        return carry

    lax.fori_loop(0, NB, add_body, 0)

    plsc.subcore_barrier()
    pltpu.sync_copy(
        deg_sh.at[pl.ds(s * TROWS, TROWS)],
        deg_hbm.at[pl.ds(c * N_ACC + s * TROWS, TROWS)],
    )


@functools.partial(
    pl.kernel,
    out_type=jax.ShapeDtypeStruct((NC * N_ACC, F), jnp.float32),
    mesh=_mesh,
    scratch_types=[
        pltpu.VMEM((NBC, B), jnp.int32),
        pltpu.VMEM((NBC, B), jnp.int32),
        pltpu.VMEM((B, F), jnp.float32),
        pltpu.VMEM((B, F), jnp.float32),
        pltpu.VMEM((B, F), jnp.float32),
        pltpu.VMEM_SHARED((N_ACC, F), jnp.float32),
        pltpu.SemaphoreType.DMA,
        pltpu.SemaphoreType.DMA,
        pltpu.SemaphoreType.DMA,
        pltpu.SemaphoreType.DMA,
        pltpu.SemaphoreType.DMA,
        pltpu.SemaphoreType.DMA,
    ],
)
def _sc_spmm(h2_hbm, ei_hbm, dest_hbm, zinit_hbm, acc_hbm,
             col_v, dest_v, gb0, gb1, gb2, acc_sh,
             gs0, gs1, gs2, ss0, ss1, ss2):
    c = lax.axis_index("c")
    s = lax.axis_index("s")
    w = c * NS + s

    pltpu.sync_copy(zinit_hbm, acc_sh.at[pl.ds(s * TROWS, TROWS)])
    plsc.subcore_barrier()

    gbuf = (gb0, gb1, gb2)
    gsem = (gs0, gs1, gs2)
    ssem = (ss0, ss1, ss2)

    def gather(j, r):
        return pltpu.make_async_copy(h2_hbm.at[col_v.at[j]], gbuf[r], gsem[r])

    def scat(j, r):
        return pltpu.make_async_copy(gbuf[r], acc_sh.at[dest_v.at[j]], ssem[r])

    def chunk(h, carry):
        pltpu.sync_copy(ei_hbm.at[1, w, h], col_v)
        pltpu.sync_copy(dest_hbm.at[w, h], dest_v)

        gather(0, 0).start()
        gather(1, 1).start()
        gather(0, 0).wait()
        scat(0, 0).start(add=True)
        gather(2, 2).start()
        gather(1, 1).wait()
        scat(1, 1).start(add=True)
        scat(0, 0).wait()
        gather(3, 0).start()
        gather(2, 2).wait()
        scat(2, 2).start(add=True)
        scat(1, 1).wait()
        gather(4, 1).start()

        def body(k, cc):
            j0 = 3 * k
            gather(j0, 0).wait()
            scat(j0, 0).start(add=True)
            scat(j0 - 1, 2).wait()
            gather(j0 + 2, 2).start()
            gather(j0 + 1, 1).wait()
            scat(j0 + 1, 1).start(add=True)
            scat(j0, 0).wait()
            gather(j0 + 3, 0).start()
            gather(j0 + 2, 2).wait()
            scat(j0 + 2, 2).start(add=True)
            scat(j0 + 1, 1).wait()
            gather(j0 + 4, 1).start()
            return cc

        lax.fori_loop(1, NBC // 3 - 1, body, 0)

        j0 = NBC - 3
        gather(j0, 0).wait()
        scat(j0, 0).start(add=True)
        scat(j0 - 1, 2).wait()
        gather(j0 + 2, 2).start()
        gather(j0 + 1, 1).wait()
        scat(j0 + 1, 1).start(add=True)
        scat(j0, 0).wait()
        gather(j0 + 2, 2).wait()
        scat(j0 + 2, 2).start(add=True)
        scat(j0 + 1, 1).wait()
        scat(j0 + 2, 2).wait()
        return carry

    lax.fori_loop(0, NCH, chunk, 0)

    plsc.subcore_barrier()
    pltpu.sync_copy(
        acc_sh.at[pl.ds(s * TROWS, TROWS)],
        acc_hbm.at[pl.ds(c * N_ACC + s * TROWS, TROWS)],
    )


_RB = 1000


def _tc_h2_body(x_ref, w_ref, deg_ref, h2_ref):
    h = jnp.dot(x_ref[...], w_ref[...], preferred_element_type=jnp.float32)
    deg = deg_ref[:, 0] + deg_ref[:, 1] + 1.0
    dis = lax.rsqrt(deg)
    h2_ref[...] = h * dis[:, None]


_tc_h2 = pl.pallas_call(
    _tc_h2_body,
    grid=(N_NODES // _RB,),
    in_specs=[
        pl.BlockSpec((_RB, F), lambda i: (i, 0)),
        pl.BlockSpec((F, F), lambda i: (0, 0)),
        pl.BlockSpec((_RB, NC), lambda i: (i, 0)),
    ],
    out_specs=pl.BlockSpec((_RB, F), lambda i: (i, 0)),
    out_shape=jax.ShapeDtypeStruct((N_NODES, F), jnp.float32),
)


def _tc_out_body(acc_ref, h2_ref, deg_ref, b_ref, o_ref):
    acc = acc_ref[0] + acc_ref[1]
    deg = deg_ref[:, 0] + deg_ref[:, 1] + 1.0
    dis = lax.rsqrt(deg)
    o_ref[...] = (acc + h2_ref[...]) * dis[:, None] + b_ref[...]


_tc_out = pl.pallas_call(
    _tc_out_body,
    grid=(N_NODES // _RB,),
    in_specs=[
        pl.BlockSpec((NC, _RB, F), lambda i: (0, i, 0)),
        pl.BlockSpec((_RB, F), lambda i: (i, 0)),
        pl.BlockSpec((_RB, NC), lambda i: (i, 0)),
        pl.BlockSpec((1, F), lambda i: (0, 0)),
    ],
    out_specs=pl.BlockSpec((_RB, F), lambda i: (i, 0)),
    out_shape=jax.ShapeDtypeStruct((N_NODES, F), jnp.float32),
)


def kernel(x, edge_index, weight, bias):
    assert x.shape == (N_NODES, F) and edge_index.shape == (2, N_EDGES)
    ei_p = jnp.concatenate([edge_index, _EI_PAD], axis=1).reshape(2, NW, NB, B)

    deg_parts, dest = _sc_deg(ei_p)
    ei5 = ei_p.reshape(2, NW, NCH, NBC, B)
    dest5 = dest.reshape(NW, NCH, NBC, B)
    deg_parts = deg_parts.reshape(NC, N_ACC)
    deg2 = jnp.stack([deg_parts[0, :N_NODES], deg_parts[1, :N_NODES]], axis=1)

    h2 = _tc_h2(x, weight, deg2)

    zinit = jnp.zeros((TROWS, F), jnp.float32)
    acc_parts = _sc_spmm(h2, ei5, dest5, zinit).reshape(NC, N_ACC, F)

    out = _tc_out(acc_parts, h2, deg2, bias.reshape(1, F))
    return out
